# baseline (speedup 1.0000x reference)
import functools

import jax
import jax.numpy as jnp
from jax import lax
from jax.experimental import pallas as pl
from jax.experimental.pallas import tpu as pltpu
from jax.experimental.pallas import tpu_sc as plsc

NUM_DOMAINS = 4
D = 1024
S = 20
B = 1024
G = 64
NB = B // G + NUM_DOMAINS
CAP = NB * G
SP = 24
BR = G * SP
L = 16
NCH = B // L
NBP = 32


def _sc_routing(ids):
    mesh = plsc.VectorSubcoreMesh(core_axis_name="c", subcore_axis_name="s")

    @functools.partial(
        pl.kernel, mesh=mesh,
        compiler_params=pltpu.CompilerParams(needs_layout_passes=False),
        out_type=[
            jax.ShapeDtypeStruct((CAP,), jnp.int32),
            jax.ShapeDtypeStruct((NBP,), jnp.int32),
            jax.ShapeDtypeStruct((NBP,), jnp.int32),
            jax.ShapeDtypeStruct((L,), jnp.int32),
        ],
        scratch_types=[
            pltpu.VMEM((B,), jnp.int32),
            pltpu.VMEM((CAP,), jnp.int32),
            pltpu.VMEM((NBP,), jnp.int32),
            pltpu.VMEM((NBP,), jnp.int32),
            pltpu.VMEM((L,), jnp.int32),
        ],
    )
    def k(ids_hbm, src_hbm, dom_hbm, vcnt_hbm, cnt_hbm,
          ids_v, src_v, dom_v, vcnt_v, cnt_v):
        wid = lax.axis_index("s") * 2 + lax.axis_index("c")

        @pl.when(wid == 0)
        def _():
            pltpu.sync_copy(ids_hbm, ids_v)
            zero = jnp.zeros((L,), jnp.int32)

            def zero_body(c, carry):
                src_v[pl.ds(c * L, L)] = zero
                return carry

            lax.fori_loop(0, CAP // L, zero_body, jnp.int32(0))

            def hist_body(c, cnts):
                v = ids_v[pl.ds(c * L, L)]
                return tuple(
                    cnts[d] + jnp.sum((v == d).astype(jnp.int32))
                    for d in range(NUM_DOMAINS))

            cnts = lax.fori_loop(
                0, NCH, hist_body,
                tuple(jnp.int32(0) for _ in range(NUM_DOMAINS)))
            cnt_up = jnp.zeros((L,), jnp.int32)
            for d in range(NUM_DOMAINS):
                cnt_up = jnp.where(lax.iota(jnp.int32, L) == d, cnts[d], cnt_up)
            cnt_v[...] = cnt_up

            offs, ends, acc = [], [], jnp.int32(0)
            for d in range(NUM_DOMAINS):
                offs.append(acc)
                acc = acc + ((cnts[d] + G - 1) // G) * G
                ends.append(acc)
            iota = lax.iota(jnp.int32, L)

            def rank_body(c, runs):
                v = ids_v[pl.ds(c * L, L)]
                dst = jnp.zeros((L,), jnp.int32)
                new_runs = []
                for d in range(NUM_DOMAINS):
                    m = v == d
                    mi = m.astype(jnp.int32)
                    ex = plsc.cumsum(mi) - mi
                    dst = jnp.where(m, offs[d] + runs[d] + ex, dst)
                    new_runs.append(runs[d] + jnp.sum(mi))
                plsc.store_scatter(src_v, [dst], iota + c * L)
                return tuple(new_runs)

            lax.fori_loop(0, NCH, rank_body,
                          tuple(jnp.int32(0) for _ in range(NUM_DOMAINS)))

            for c in range(NBP // L):
                kg = (iota + c * L) * G
                domv = jnp.zeros((L,), jnp.int32)
                for d in range(NUM_DOMAINS - 1):
                    domv = domv + (kg >= ends[d]).astype(jnp.int32)
                csel = jnp.full((L,), cnts[NUM_DOMAINS - 1], jnp.int32)
                osel = jnp.full((L,), offs[NUM_DOMAINS - 1], jnp.int32)
                for d in range(NUM_DOMAINS - 2, -1, -1):
                    cond = domv == d
                    csel = jnp.where(cond, cnts[d], csel)
                    osel = jnp.where(cond, offs[d], osel)
                vc = jnp.clip(csel - (kg - osel), 0, G)
                dom_v[pl.ds(c * L, L)] = domv
                vcnt_v[pl.ds(c * L, L)] = vc

            pltpu.sync_copy(src_v, src_hbm)
            pltpu.sync_copy(dom_v, dom_hbm)
            pltpu.sync_copy(vcnt_v, vcnt_hbm)
            pltpu.sync_copy(cnt_v, cnt_hbm)

    return k(ids)


def _routing(domain_ids):
    ids = domain_ids.astype(jnp.int32)
    oh = (ids[:, None] == jnp.arange(NUM_DOMAINS, dtype=jnp.int32)[None, :])
    oh = oh.astype(jnp.int32)
    cnt = jnp.sum(oh, axis=0)
    pc = ((cnt + G - 1) // G) * G
    ends = jnp.cumsum(pc)
    off = ends - pc
    rank = jnp.cumsum(oh, axis=0) - oh
    myrank = jnp.take_along_axis(rank, ids[:, None], axis=1)[:, 0]
    dst = off[ids] + myrank
    src = jnp.zeros((CAP,), jnp.int32).at[dst].set(
        jnp.arange(B, dtype=jnp.int32))
    kg = jnp.arange(NB, dtype=jnp.int32) * G
    dom = jnp.sum((kg[:, None] >= ends[None, :]).astype(jnp.int32), axis=1)
    dom = jnp.minimum(dom, NUM_DOMAINS - 1)
    vcnt = jnp.clip(cnt[dom] - (kg - off[dom]), 0, G)
    return src, dom, vcnt, cnt


def _fused_body(src_ref, dom_ref, vcnt_ref, cnt_ref,
                feat_ref, w_ref, a_ref,
                out_ref, loss_ref,
                xbuf, ybuf, xbig, acc_ref, gsem, ssem):
    k = pl.program_id(0)
    nb = pl.num_programs(0)

    def issue_gather(kk, slot):
        for j in range(G):
            pltpu.make_async_copy(
                feat_ref.at[src_ref[kk * G + j]],
                xbuf.at[slot, j],
                gsem.at[slot],
            ).start()

    @pl.when(k == 0)
    def _prime():
        issue_gather(0, 0)

    @pl.when(k + 1 < nb)
    def _prefetch():
        issue_gather(k + 1, jax.lax.rem(k + 1, 2))

    slot = jax.lax.rem(k, 2)

    for j in range(G):
        pltpu.make_async_copy(
            feat_ref.at[src_ref[k * G + j]], xbuf.at[slot, j], gsem.at[slot]
        ).wait()
    for j in range(G):
        xbig[pl.ds(j * SP, S), :] = xbuf[slot, j]

    w = w_ref[0]
    y = jax.lax.dot_general(xbig[...], w, (((1,), (1,)), ((), ())),
                            preferred_element_type=jnp.float32)

    @pl.when(k >= 2)
    def _drain_prev():
        kk = k - 2
        for j in range(G):
            @pl.when(j < vcnt_ref[kk])
            def _():
                pltpu.make_async_copy(
                    ybuf.at[slot, j], out_ref.at[src_ref[kk * G + j]],
                    ssem.at[slot],
                ).wait()

    @pl.when(k == 0)
    def _init_acc():
        for i in range(NUM_DOMAINS):
            acc_ref[i] = 0.0

    vcnt = vcnt_ref[k]
    for j in range(G):
        ybuf[slot, j] = y[j * SP:j * SP + S, :]

    row = jax.lax.broadcasted_iota(jnp.int32, (BR, D), 0)
    rmask = jnp.logical_and(jax.lax.rem(row, SP) < S, row < vcnt * SP)
    dif = y - a_ref[...]
    sq = jnp.sum(jnp.where(rmask, dif * dif, jnp.float32(0.0)))
    dom = dom_ref[k]
    acc_ref[dom] = acc_ref[dom] + sq

    for j in range(G):
        @pl.when(j < vcnt)
        def _():
            pltpu.make_async_copy(
                ybuf.at[slot, j],
                out_ref.at[src_ref[k * G + j]],
                ssem.at[slot],
            ).start()

    @pl.when(k == nb - 1)
    def _finish():
        for kk in (nb - 2, nb - 1):
            for j in range(G):
                @pl.when(j < vcnt_ref[kk])
                def _():
                    pltpu.make_async_copy(
                        ybuf.at[jax.lax.rem(kk, 2), j],
                        out_ref.at[src_ref[kk * G + j]],
                        ssem.at[jax.lax.rem(kk, 2)],
                    ).wait()
        total = jnp.float32(0.0)
        for i in range(NUM_DOMAINS):
            c = cnt_ref[i]
            denom = jnp.maximum(c, 1).astype(jnp.float32) * jnp.float32(S * D)
            total = total + jnp.where(c > 0, acc_ref[i] / denom,
                                      jnp.float32(0.0))
        loss_ref[0, 0] = total / jnp.float32(NUM_DOMAINS)


@jax.jit
def kernel(features, domain_ids, anchor, Ws):
    src, dom48, vcnt48, cnt16 = _sc_routing(domain_ids.astype(jnp.int32))
    dom = dom48[:NB]
    vcnt = vcnt48[:NB]
    cnt = cnt16[:NUM_DOMAINS]

    pallas = pl.pallas_call(
        _fused_body,
        grid_spec=pltpu.PrefetchScalarGridSpec(
            num_scalar_prefetch=4,
            grid=(NB,),
            in_specs=[
                pl.BlockSpec(memory_space=pltpu.MemorySpace.HBM),
                pl.BlockSpec((1, D, D),
                             lambda k, src, dom, vcnt, cnt: (dom[k], 0, 0)),
                pl.BlockSpec((BR, D),
                             lambda k, src, dom, vcnt, cnt: (0, 0)),
            ],
            out_specs=[
                pl.BlockSpec(memory_space=pltpu.MemorySpace.HBM),
                pl.BlockSpec(memory_space=pltpu.MemorySpace.SMEM),
            ],
            scratch_shapes=[
                pltpu.VMEM((2, G, S, D), jnp.float32),
                pltpu.VMEM((2, G, S, D), jnp.float32),
                pltpu.VMEM((BR, D), jnp.float32),
                pltpu.SMEM((NUM_DOMAINS,), jnp.float32),
                pltpu.SemaphoreType.DMA((2,)),
                pltpu.SemaphoreType.DMA((2,)),
            ],
        ),
        out_shape=[
            jax.ShapeDtypeStruct((B, S, D), jnp.float32),
            jax.ShapeDtypeStruct((1, 1), jnp.float32),
        ],
        compiler_params=pltpu.CompilerParams(
            dimension_semantics=("arbitrary",),
            vmem_limit_bytes=100 * 1024 * 1024),
    )
    a_pad = jnp.concatenate(
        [anchor[0], jnp.zeros((SP - S, D), jnp.float32)], axis=0)
    anchor_rep = jnp.tile(a_pad, (G, 1))
    projected, loss = pallas(src, dom, vcnt, cnt, features, Ws, anchor_rep)

    return projected, loss.reshape(())


# speedup vs baseline: 1.0381x; 1.0381x over previous
import functools

import jax
import jax.numpy as jnp
from jax import lax
from jax.experimental import pallas as pl
from jax.experimental.pallas import tpu as pltpu
from jax.experimental.pallas import tpu_sc as plsc

NUM_DOMAINS = 4
D = 1024
S = 20
B = 1024
G = 32
NB = B // G + NUM_DOMAINS
CAP = NB * G
SP = 24
BR = G * SP
L = 16
NCH = B // L
NBP = 48


def _sc_routing(ids):
    mesh = plsc.VectorSubcoreMesh(core_axis_name="c", subcore_axis_name="s")

    @functools.partial(
        pl.kernel, mesh=mesh,
        compiler_params=pltpu.CompilerParams(needs_layout_passes=False),
        out_type=[
            jax.ShapeDtypeStruct((CAP,), jnp.int32),
            jax.ShapeDtypeStruct((NBP,), jnp.int32),
            jax.ShapeDtypeStruct((NBP,), jnp.int32),
            jax.ShapeDtypeStruct((L,), jnp.int32),
        ],
        scratch_types=[
            pltpu.VMEM((B,), jnp.int32),
            pltpu.VMEM((CAP,), jnp.int32),
            pltpu.VMEM((NBP,), jnp.int32),
            pltpu.VMEM((NBP,), jnp.int32),
            pltpu.VMEM((L,), jnp.int32),
        ],
    )
    def k(ids_hbm, src_hbm, dom_hbm, vcnt_hbm, cnt_hbm,
          ids_v, src_v, dom_v, vcnt_v, cnt_v):
        wid = lax.axis_index("s") * 2 + lax.axis_index("c")

        @pl.when(wid == 0)
        def _():
            pltpu.sync_copy(ids_hbm, ids_v)
            zero = jnp.zeros((L,), jnp.int32)

            def zero_body(c, carry):
                src_v[pl.ds(c * L, L)] = zero
                return carry

            lax.fori_loop(0, CAP // L, zero_body, jnp.int32(0))

            def hist_body(c, cnts):
                v = ids_v[pl.ds(c * L, L)]
                return tuple(
                    cnts[d] + jnp.sum((v == d).astype(jnp.int32))
                    for d in range(NUM_DOMAINS))

            cnts = lax.fori_loop(
                0, NCH, hist_body,
                tuple(jnp.int32(0) for _ in range(NUM_DOMAINS)))
            cnt_up = jnp.zeros((L,), jnp.int32)
            for d in range(NUM_DOMAINS):
                cnt_up = jnp.where(lax.iota(jnp.int32, L) == d, cnts[d], cnt_up)
            cnt_v[...] = cnt_up

            offs, ends, acc = [], [], jnp.int32(0)
            for d in range(NUM_DOMAINS):
                offs.append(acc)
                acc = acc + ((cnts[d] + G - 1) // G) * G
                ends.append(acc)
            iota = lax.iota(jnp.int32, L)

            def rank_body(c, runs):
                v = ids_v[pl.ds(c * L, L)]
                dst = jnp.zeros((L,), jnp.int32)
                new_runs = []
                for d in range(NUM_DOMAINS):
                    m = v == d
                    mi = m.astype(jnp.int32)
                    ex = plsc.cumsum(mi) - mi
                    dst = jnp.where(m, offs[d] + runs[d] + ex, dst)
                    new_runs.append(runs[d] + jnp.sum(mi))
                plsc.store_scatter(src_v, [dst], iota + c * L)
                return tuple(new_runs)

            lax.fori_loop(0, NCH, rank_body,
                          tuple(jnp.int32(0) for _ in range(NUM_DOMAINS)))

            for c in range(NBP // L):
                kg = (iota + c * L) * G
                domv = jnp.zeros((L,), jnp.int32)
                for d in range(NUM_DOMAINS - 1):
                    domv = domv + (kg >= ends[d]).astype(jnp.int32)
                csel = jnp.full((L,), cnts[NUM_DOMAINS - 1], jnp.int32)
                osel = jnp.full((L,), offs[NUM_DOMAINS - 1], jnp.int32)
                for d in range(NUM_DOMAINS - 2, -1, -1):
                    cond = domv == d
                    csel = jnp.where(cond, cnts[d], csel)
                    osel = jnp.where(cond, offs[d], osel)
                vc = jnp.clip(csel - (kg - osel), 0, G)
                dom_v[pl.ds(c * L, L)] = domv
                vcnt_v[pl.ds(c * L, L)] = vc

            pltpu.sync_copy(src_v, src_hbm)
            pltpu.sync_copy(dom_v, dom_hbm)
            pltpu.sync_copy(vcnt_v, vcnt_hbm)
            pltpu.sync_copy(cnt_v, cnt_hbm)

    return k(ids)


def _routing(domain_ids):
    ids = domain_ids.astype(jnp.int32)
    oh = (ids[:, None] == jnp.arange(NUM_DOMAINS, dtype=jnp.int32)[None, :])
    oh = oh.astype(jnp.int32)
    cnt = jnp.sum(oh, axis=0)
    pc = ((cnt + G - 1) // G) * G
    ends = jnp.cumsum(pc)
    off = ends - pc
    rank = jnp.cumsum(oh, axis=0) - oh
    myrank = jnp.take_along_axis(rank, ids[:, None], axis=1)[:, 0]
    dst = off[ids] + myrank
    src = jnp.zeros((CAP,), jnp.int32).at[dst].set(
        jnp.arange(B, dtype=jnp.int32))
    kg = jnp.arange(NB, dtype=jnp.int32) * G
    dom = jnp.sum((kg[:, None] >= ends[None, :]).astype(jnp.int32), axis=1)
    dom = jnp.minimum(dom, NUM_DOMAINS - 1)
    vcnt = jnp.clip(cnt[dom] - (kg - off[dom]), 0, G)
    return src, dom, vcnt, cnt


def _fused_body(src_ref, dom_ref, vcnt_ref, cnt_ref,
                feat_ref, w_ref, a_ref,
                out_ref, loss_ref,
                xbuf, ybuf, xbig, acc_ref, gsem, ssem):
    k = pl.program_id(0)
    nb = pl.num_programs(0)

    def issue_gather(kk, slot):
        for j in range(G):
            pltpu.make_async_copy(
                feat_ref.at[src_ref[kk * G + j]],
                xbuf.at[slot, j],
                gsem.at[slot],
            ).start()

    @pl.when(k == 0)
    def _prime():
        issue_gather(0, 0)

    @pl.when(k + 1 < nb)
    def _prefetch():
        issue_gather(k + 1, jax.lax.rem(k + 1, 2))

    slot = jax.lax.rem(k, 2)

    for j in range(G):
        pltpu.make_async_copy(
            feat_ref.at[src_ref[k * G + j]], xbuf.at[slot, j], gsem.at[slot]
        ).wait()
    for j in range(G):
        xbig[pl.ds(j * SP, S), :] = xbuf[slot, j]

    w = w_ref[0]
    y = jax.lax.dot_general(xbig[...], w, (((1,), (1,)), ((), ())),
                            preferred_element_type=jnp.float32)

    @pl.when(k >= 2)
    def _drain_prev():
        kk = k - 2
        for j in range(G):
            @pl.when(j < vcnt_ref[kk])
            def _():
                pltpu.make_async_copy(
                    ybuf.at[slot, j], out_ref.at[src_ref[kk * G + j]],
                    ssem.at[slot],
                ).wait()

    @pl.when(k == 0)
    def _init_acc():
        for i in range(NUM_DOMAINS):
            acc_ref[i] = 0.0

    vcnt = vcnt_ref[k]
    for j in range(G):
        ybuf[slot, j] = y[j * SP:j * SP + S, :]

    row = jax.lax.broadcasted_iota(jnp.int32, (BR, D), 0)
    rmask = jnp.logical_and(jax.lax.rem(row, SP) < S, row < vcnt * SP)
    dif = y - a_ref[...]
    sq = jnp.sum(jnp.where(rmask, dif * dif, jnp.float32(0.0)))
    dom = dom_ref[k]
    acc_ref[dom] = acc_ref[dom] + sq

    for j in range(G):
        @pl.when(j < vcnt)
        def _():
            pltpu.make_async_copy(
                ybuf.at[slot, j],
                out_ref.at[src_ref[k * G + j]],
                ssem.at[slot],
            ).start()

    @pl.when(k == nb - 1)
    def _finish():
        for kk in (nb - 2, nb - 1):
            for j in range(G):
                @pl.when(j < vcnt_ref[kk])
                def _():
                    pltpu.make_async_copy(
                        ybuf.at[jax.lax.rem(kk, 2), j],
                        out_ref.at[src_ref[kk * G + j]],
                        ssem.at[jax.lax.rem(kk, 2)],
                    ).wait()
        total = jnp.float32(0.0)
        for i in range(NUM_DOMAINS):
            c = cnt_ref[i]
            denom = jnp.maximum(c, 1).astype(jnp.float32) * jnp.float32(S * D)
            total = total + jnp.where(c > 0, acc_ref[i] / denom,
                                      jnp.float32(0.0))
        loss_ref[0, 0] = total / jnp.float32(NUM_DOMAINS)


@jax.jit
def kernel(features, domain_ids, anchor, Ws):
    src, dom48, vcnt48, cnt16 = _sc_routing(domain_ids.astype(jnp.int32))
    dom = dom48[:NB]
    vcnt = vcnt48[:NB]
    cnt = cnt16[:NUM_DOMAINS]

    pallas = pl.pallas_call(
        _fused_body,
        grid_spec=pltpu.PrefetchScalarGridSpec(
            num_scalar_prefetch=4,
            grid=(NB,),
            in_specs=[
                pl.BlockSpec(memory_space=pltpu.MemorySpace.HBM),
                pl.BlockSpec((1, D, D),
                             lambda k, src, dom, vcnt, cnt: (dom[k], 0, 0)),
                pl.BlockSpec((BR, D),
                             lambda k, src, dom, vcnt, cnt: (0, 0)),
            ],
            out_specs=[
                pl.BlockSpec(memory_space=pltpu.MemorySpace.HBM),
                pl.BlockSpec(memory_space=pltpu.MemorySpace.SMEM),
            ],
            scratch_shapes=[
                pltpu.VMEM((2, G, S, D), jnp.float32),
                pltpu.VMEM((2, G, S, D), jnp.float32),
                pltpu.VMEM((BR, D), jnp.float32),
                pltpu.SMEM((NUM_DOMAINS,), jnp.float32),
                pltpu.SemaphoreType.DMA((2,)),
                pltpu.SemaphoreType.DMA((2,)),
            ],
        ),
        out_shape=[
            jax.ShapeDtypeStruct((B, S, D), jnp.float32),
            jax.ShapeDtypeStruct((1, 1), jnp.float32),
        ],
        compiler_params=pltpu.CompilerParams(
            dimension_semantics=("arbitrary",)),
    )
    a_pad = jnp.concatenate(
        [anchor[0], jnp.zeros((SP - S, D), jnp.float32)], axis=0)
    anchor_rep = jnp.tile(a_pad, (G, 1))
    projected, loss = pallas(src, dom, vcnt, cnt, features, Ws, anchor_rep)

    return projected, loss.reshape(())


# speedup vs baseline: 1.0746x; 1.0351x over previous
import functools

import jax
import jax.numpy as jnp
from jax import lax
from jax.experimental import pallas as pl
from jax.experimental.pallas import tpu as pltpu
from jax.experimental.pallas import tpu_sc as plsc

NUM_DOMAINS = 4
D = 1024
S = 20
B = 1024
G = 32
NB = B // G + NUM_DOMAINS
CAP = NB * G
SP = 24
BR = G * SP
L = 16
NCH = B // L
NBP = 48


def _sc_routing(ids):
    mesh = plsc.VectorSubcoreMesh(core_axis_name="c", subcore_axis_name="s")

    @functools.partial(
        pl.kernel, mesh=mesh,
        compiler_params=pltpu.CompilerParams(needs_layout_passes=False),
        out_type=[
            jax.ShapeDtypeStruct((CAP,), jnp.int32),
            jax.ShapeDtypeStruct((NBP,), jnp.int32),
            jax.ShapeDtypeStruct((NBP,), jnp.int32),
            jax.ShapeDtypeStruct((L,), jnp.int32),
        ],
        scratch_types=[
            pltpu.VMEM((B,), jnp.int32),
            pltpu.VMEM((CAP,), jnp.int32),
            pltpu.VMEM((NBP,), jnp.int32),
            pltpu.VMEM((NBP,), jnp.int32),
            pltpu.VMEM((L,), jnp.int32),
        ],
    )
    def k(ids_hbm, src_hbm, dom_hbm, vcnt_hbm, cnt_hbm,
          ids_v, src_v, dom_v, vcnt_v, cnt_v):
        wid = lax.axis_index("s") * 2 + lax.axis_index("c")

        @pl.when(wid == 0)
        def _():
            pltpu.sync_copy(ids_hbm, ids_v)
            zero = jnp.zeros((L,), jnp.int32)

            def zero_body(c, carry):
                src_v[pl.ds(c * L, L)] = zero
                return carry

            lax.fori_loop(0, CAP // L, zero_body, jnp.int32(0))

            def hist_body(c, cnts):
                v = ids_v[pl.ds(c * L, L)]
                return tuple(
                    cnts[d] + jnp.sum((v == d).astype(jnp.int32))
                    for d in range(NUM_DOMAINS))

            cnts = lax.fori_loop(
                0, NCH, hist_body,
                tuple(jnp.int32(0) for _ in range(NUM_DOMAINS)))
            cnt_up = jnp.zeros((L,), jnp.int32)
            for d in range(NUM_DOMAINS):
                cnt_up = jnp.where(lax.iota(jnp.int32, L) == d, cnts[d], cnt_up)
            cnt_v[...] = cnt_up

            offs, ends, acc = [], [], jnp.int32(0)
            for d in range(NUM_DOMAINS):
                offs.append(acc)
                acc = acc + ((cnts[d] + G - 1) // G) * G
                ends.append(acc)
            iota = lax.iota(jnp.int32, L)

            def rank_body(c, runs):
                v = ids_v[pl.ds(c * L, L)]
                dst = jnp.zeros((L,), jnp.int32)
                new_runs = []
                for d in range(NUM_DOMAINS):
                    m = v == d
                    mi = m.astype(jnp.int32)
                    ex = plsc.cumsum(mi) - mi
                    dst = jnp.where(m, offs[d] + runs[d] + ex, dst)
                    new_runs.append(runs[d] + jnp.sum(mi))
                plsc.store_scatter(src_v, [dst], iota + c * L)
                return tuple(new_runs)

            lax.fori_loop(0, NCH, rank_body,
                          tuple(jnp.int32(0) for _ in range(NUM_DOMAINS)))

            for c in range(NBP // L):
                kg = (iota + c * L) * G
                domv = jnp.zeros((L,), jnp.int32)
                for d in range(NUM_DOMAINS - 1):
                    domv = domv + (kg >= ends[d]).astype(jnp.int32)
                csel = jnp.full((L,), cnts[NUM_DOMAINS - 1], jnp.int32)
                osel = jnp.full((L,), offs[NUM_DOMAINS - 1], jnp.int32)
                for d in range(NUM_DOMAINS - 2, -1, -1):
                    cond = domv == d
                    csel = jnp.where(cond, cnts[d], csel)
                    osel = jnp.where(cond, offs[d], osel)
                vc = jnp.clip(csel - (kg - osel), 0, G)
                dom_v[pl.ds(c * L, L)] = domv
                vcnt_v[pl.ds(c * L, L)] = vc

            pltpu.sync_copy(src_v, src_hbm)
            pltpu.sync_copy(dom_v, dom_hbm)
            pltpu.sync_copy(vcnt_v, vcnt_hbm)
            pltpu.sync_copy(cnt_v, cnt_hbm)

    return k(ids)


def _fused_body(src_ref, dom_ref, vcnt_ref, cnt_ref,
                feat_ref, w_ref, a_ref,
                out_ref, loss_ref,
                xbuf, ybuf, xbig, acc_ref, gsem, ssem):
    k = pl.program_id(0)
    nb = pl.num_programs(0)

    def issue_gather(kk, slot):
        for j in range(G):
            pltpu.make_async_copy(
                feat_ref.at[src_ref[kk * G + j]],
                xbuf.at[slot, j],
                gsem.at[slot],
            ).start()

    @pl.when(k == 0)
    def _prime():
        issue_gather(0, 0)

    @pl.when(k + 1 < nb)
    def _prefetch():
        issue_gather(k + 1, jax.lax.rem(k + 1, 2))

    slot = jax.lax.rem(k, 2)

    for j in range(G):
        pltpu.make_async_copy(
            feat_ref.at[src_ref[k * G + j]], xbuf.at[slot, j], gsem.at[slot]
        ).wait()
    for j in range(G):
        xbig[pl.ds(j * SP, S), :] = xbuf[slot, j]

    w = w_ref[0]
    y = jax.lax.dot_general(xbig[...], w, (((1,), (1,)), ((), ())),
                            preferred_element_type=jnp.float32)

    @pl.when(k >= 2)
    def _drain_prev():
        kk = k - 2
        for j in range(G):
            @pl.when(j < vcnt_ref[kk])
            def _():
                pltpu.make_async_copy(
                    ybuf.at[slot, j], out_ref.at[src_ref[kk * G + j]],
                    ssem.at[slot],
                ).wait()

    @pl.when(k == 0)
    def _init_acc():
        for i in range(NUM_DOMAINS):
            acc_ref[i] = 0.0

    vcnt = vcnt_ref[k]
    anc = a_ref[...]
    sq = jnp.float32(0.0)
    for j in range(G):
        val = y[j * SP:j * SP + S, :]
        ybuf[slot, j] = val
        dif = val - anc
        sq = sq + jnp.where(j < vcnt, jnp.sum(dif * dif), jnp.float32(0.0))
    dom = dom_ref[k]
    acc_ref[dom] = acc_ref[dom] + sq

    for j in range(G):
        @pl.when(j < vcnt)
        def _():
            pltpu.make_async_copy(
                ybuf.at[slot, j],
                out_ref.at[src_ref[k * G + j]],
                ssem.at[slot],
            ).start()

    @pl.when(k == nb - 1)
    def _finish():
        for kk in (nb - 2, nb - 1):
            for j in range(G):
                @pl.when(j < vcnt_ref[kk])
                def _():
                    pltpu.make_async_copy(
                        ybuf.at[jax.lax.rem(kk, 2), j],
                        out_ref.at[src_ref[kk * G + j]],
                        ssem.at[jax.lax.rem(kk, 2)],
                    ).wait()
        total = jnp.float32(0.0)
        for i in range(NUM_DOMAINS):
            c = cnt_ref[i]
            denom = jnp.maximum(c, 1).astype(jnp.float32) * jnp.float32(S * D)
            total = total + jnp.where(c > 0, acc_ref[i] / denom,
                                      jnp.float32(0.0))
        loss_ref[0, 0] = total / jnp.float32(NUM_DOMAINS)


@jax.jit
def kernel(features, domain_ids, anchor, Ws):
    src, dom48, vcnt48, cnt16 = _sc_routing(domain_ids.astype(jnp.int32))
    dom = dom48[:NB]
    vcnt = vcnt48[:NB]
    cnt = cnt16[:NUM_DOMAINS]

    projected, loss = pl.pallas_call(
        _fused_body,
        grid_spec=pltpu.PrefetchScalarGridSpec(
            num_scalar_prefetch=4,
            grid=(NB,),
            in_specs=[
                pl.BlockSpec(memory_space=pltpu.MemorySpace.HBM),
                pl.BlockSpec((1, D, D),
                             lambda k, src, dom, vcnt, cnt: (dom[k], 0, 0)),
                pl.BlockSpec((S, D),
                             lambda k, src, dom, vcnt, cnt: (0, 0)),
            ],
            out_specs=[
                pl.BlockSpec(memory_space=pltpu.MemorySpace.HBM),
                pl.BlockSpec(memory_space=pltpu.MemorySpace.SMEM),
            ],
            scratch_shapes=[
                pltpu.VMEM((2, G, S, D), jnp.float32),
                pltpu.VMEM((2, G, S, D), jnp.float32),
                pltpu.VMEM((BR, D), jnp.float32),
                pltpu.SMEM((NUM_DOMAINS,), jnp.float32),
                pltpu.SemaphoreType.DMA((2,)),
                pltpu.SemaphoreType.DMA((2,)),
            ],
        ),
        out_shape=[
            jax.ShapeDtypeStruct((B, S, D), jnp.float32),
            jax.ShapeDtypeStruct((1, 1), jnp.float32),
        ],
        compiler_params=pltpu.CompilerParams(
            dimension_semantics=("arbitrary",)),
    )(src, dom, vcnt, cnt, features, Ws, anchor[0])

    return projected, loss.reshape(())
